# baseline (speedup 1.0000x reference)
import functools
import math

import jax
import jax.numpy as jnp
import numpy as np
from jax.experimental import pallas as pl
from jax.experimental.pallas import tpu as pltpu

_FACTOR = 5
_NEG_INF = float("-inf")


@functools.lru_cache(maxsize=None)
def _sample_counts_np(l_q: int, l_k: int, u: int):
    with jax.ensure_compile_time_eval():
        idx = jax.random.randint(jax.random.key(42), (l_q, u), 0, l_k)
        idx_np = np.asarray(idx)
    counts = np.zeros((l_q, l_k), np.float32)
    np.add.at(counts, (np.arange(l_q)[:, None], idx_np), 1.0)
    return counts


def _sample_counts(l_q: int, l_k: int, u: int):
    try:
        return jnp.asarray(_sample_counts_np(l_q, l_k, u))
    except Exception:
        idx = jax.random.randint(jax.random.key(42), (l_q, u), 0, l_k)
        return jnp.zeros((l_q, l_k), jnp.float32).at[
            jnp.arange(l_q)[:, None], idx].add(1.0)


def _attn_kernel(c_ref, q_ref, k_ref, v_ref, o_ref, m_ref, oh_ref,
                 *, l_q, l_k, d, n_top, qt, scale):
    q = q_ref[:, :]
    k = k_ref[:, :]
    v = v_ref[:, :]

    n_tiles = l_q // qt
    for t in range(n_tiles):
        q_t = q[t * qt:(t + 1) * qt, :]
        c_t = c_ref[t * qt:(t + 1) * qt, :]
        s_t = jax.lax.dot_general(
            q_t, k, (((1,), (1,)), ((), ())),
            preferred_element_type=jnp.float32)
        kc_t = jax.lax.dot_general(
            c_t, k, (((1,), (0,)), ((), ())),
            preferred_element_type=jnp.float32)
        s_sum = jnp.sum(q_t * kc_t, axis=1)
        s_max = jnp.max(jnp.where(c_t > 0.0, s_t, _NEG_INF), axis=1)
        m_ref[t, :] = s_max - s_sum / l_k

    m_pos = (jax.lax.broadcasted_iota(jnp.int32, (n_tiles, qt), 0) * qt
             + jax.lax.broadcasted_iota(jnp.int32, (n_tiles, qt), 1))
    col_iota = jax.lax.broadcasted_iota(jnp.int32, (1, l_q), 1)

    def body(i, _):
        m = m_ref[:]
        m_max = jnp.max(m)
        li = jnp.min(jnp.where(m == m_max, m_pos, l_q))
        oh_ref[pl.ds(i, 1), :] = (col_iota == li).astype(jnp.float32)
        m_ref[:] = jnp.where(m_pos == li, _NEG_INF, m)
        return 0

    jax.lax.fori_loop(0, n_top, body, 0)
    oh = oh_ref[:]

    q_red = jax.lax.dot_general(
        oh, q, (((1,), (0,)), ((), ())),
        preferred_element_type=jnp.float32)
    scores = jax.lax.dot_general(
        q_red, k, (((1,), (1,)), ((), ())),
        preferred_element_type=jnp.float32) * scale
    s_max = jnp.max(scores, axis=1, keepdims=True)
    e = jnp.exp(scores - s_max)
    attn = e / jnp.sum(e, axis=1, keepdims=True)
    upd = jax.lax.dot_general(
        attn, v, (((1,), (0,)), ((), ())),
        preferred_element_type=jnp.float32)

    v_mean = jnp.mean(v, axis=0, keepdims=True)
    scattered = jax.lax.dot_general(
        oh, upd, (((0,), (0,)), ((), ())),
        preferred_element_type=jnp.float32)
    row_sel = jnp.sum(oh, axis=0)[:, None]
    out = jnp.where(row_sel > 0.5, scattered,
                    jnp.broadcast_to(v_mean, (l_q, d)))
    o_ref[:, :] = out


def kernel(queries, keys, values):
    b, l_q, h, d = queries.shape
    l_k = keys.shape[1]
    u_part = min(_FACTOR * int(math.ceil(math.log(l_k))), l_k)
    u = min(_FACTOR * int(math.ceil(math.log(l_q))), l_q)
    counts = jnp.asarray(_sample_counts(l_q, l_k, u))
    scale = 1.0 / math.sqrt(d)
    qt = 256
    n_tiles = l_q // qt

    q_t = jnp.transpose(queries, (0, 2, 1, 3)).reshape(b * h, l_q, d)
    k_t = jnp.transpose(keys, (0, 2, 1, 3)).reshape(b * h, l_k, d)
    v_t = jnp.transpose(values, (0, 2, 1, 3)).reshape(b * h, l_k, d)

    out = pl.pallas_call(
        functools.partial(_attn_kernel, l_q=l_q, l_k=l_k, d=d,
                          n_top=u_part, qt=qt, scale=scale),
        grid=(b * h,),
        in_specs=[
            pl.BlockSpec((l_q, l_k), lambda g: (0, 0)),
            pl.BlockSpec((None, l_q, d), lambda g: (g, 0, 0)),
            pl.BlockSpec((None, l_k, d), lambda g: (g, 0, 0)),
            pl.BlockSpec((None, l_k, d), lambda g: (g, 0, 0)),
        ],
        out_specs=pl.BlockSpec((None, l_q, d), lambda g: (g, 0, 0)),
        out_shape=jax.ShapeDtypeStruct((b * h, l_q, d), jnp.float32),
        scratch_shapes=[
            pltpu.VMEM((n_tiles, qt), jnp.float32),
            pltpu.VMEM((u_part, l_q), jnp.float32),
        ],
        compiler_params=pltpu.CompilerParams(
            dimension_semantics=("arbitrary",)),
    )(counts, q_t, k_t, v_t)
    return jnp.transpose(out.reshape(b, h, l_q, d), (0, 2, 1, 3))


# speedup vs baseline: 2.5001x; 2.5001x over previous
import functools
import math

import jax
import jax.numpy as jnp
import numpy as np
from jax.experimental import pallas as pl
from jax.experimental.pallas import tpu as pltpu

_FACTOR = 5
_NEG_INF = float("-inf")


@functools.lru_cache(maxsize=None)
def _sample_counts_np(l_q: int, l_k: int, u: int):
    with jax.ensure_compile_time_eval():
        idx = jax.random.randint(jax.random.key(42), (l_q, u), 0, l_k)
        idx_np = np.asarray(idx)
    counts = np.zeros((l_q, l_k), np.float32)
    np.add.at(counts, (np.arange(l_q)[:, None], idx_np), 1.0)
    return counts


def _sample_counts(l_q: int, l_k: int, u: int):
    try:
        return jnp.asarray(_sample_counts_np(l_q, l_k, u))
    except Exception:
        idx = jax.random.randint(jax.random.key(42), (l_q, u), 0, l_k)
        return jnp.zeros((l_q, l_k), jnp.float32).at[
            jnp.arange(l_q)[:, None], idx].add(1.0)


def _measure_kernel(c_ref, q_ref, k_ref, m_ref, *, l_k, qt):
    q = q_ref[:, :]
    k = k_ref[:, :]
    n_tiles = q.shape[0] // qt
    for t in range(n_tiles):
        q_t = q[t * qt:(t + 1) * qt, :]
        c_t = c_ref[t * qt:(t + 1) * qt, :]
        s_t = jax.lax.dot_general(
            q_t, k, (((1,), (1,)), ((), ())),
            preferred_element_type=jnp.float32)
        kc_t = jax.lax.dot_general(
            c_t, k, (((1,), (0,)), ((), ())),
            preferred_element_type=jnp.float32)
        s_sum = jnp.sum(q_t * kc_t, axis=1)
        s_max = jnp.max(jnp.where(c_t > 0.0, s_t, _NEG_INF), axis=1)
        m_ref[t, :] = s_max - s_sum / l_k


def _topk_kernel(m_in_ref, sel_ref, m_ref, *, n_top, l_q):
    col = jax.lax.broadcasted_iota(jnp.int32, m_in_ref.shape, 1)
    m_ref[:] = m_in_ref[:]
    sel_ref[:] = jnp.full(m_in_ref.shape, -1, jnp.int32)

    def body(i, _):
        m = m_ref[:]
        mx = jnp.max(m, axis=1, keepdims=True)
        li = jnp.min(jnp.where(m == mx, col, l_q), axis=1, keepdims=True)
        hit = col == li
        sel_ref[:] = jnp.where(hit, i, sel_ref[:])
        m_ref[:] = jnp.where(hit, _NEG_INF, m)
        return 0

    jax.lax.fori_loop(0, n_top, body, 0)


def _attn_kernel(sel_ref, q_ref, k_ref, v_ref, o_ref, *, l_q, d, n_top,
                 scale):
    q = q_ref[:, :]
    k = k_ref[:, :]
    v = v_ref[:, :]

    row_iota = jax.lax.broadcasted_iota(jnp.int32, (n_top, l_q), 0)
    oh = (row_iota == sel_ref[:, :]).astype(jnp.float32)

    q_red = jax.lax.dot_general(
        oh, q, (((1,), (0,)), ((), ())),
        preferred_element_type=jnp.float32)
    scores = jax.lax.dot_general(
        q_red, k, (((1,), (1,)), ((), ())),
        preferred_element_type=jnp.float32) * scale
    s_max = jnp.max(scores, axis=1, keepdims=True)
    e = jnp.exp(scores - s_max)
    attn = e / jnp.sum(e, axis=1, keepdims=True)
    upd = jax.lax.dot_general(
        attn, v, (((1,), (0,)), ((), ())),
        preferred_element_type=jnp.float32)

    v_mean = jnp.mean(v, axis=0, keepdims=True)
    scattered = jax.lax.dot_general(
        oh, upd, (((0,), (0,)), ((), ())),
        preferred_element_type=jnp.float32)
    row_sel = jnp.sum(oh, axis=0)[:, None]
    o_ref[:, :] = jnp.where(row_sel > 0.5, scattered,
                            jnp.broadcast_to(v_mean, (l_q, d)))


def kernel(queries, keys, values):
    b, l_q, h, d = queries.shape
    l_k = keys.shape[1]
    u_part = min(_FACTOR * int(math.ceil(math.log(l_k))), l_k)
    u = min(_FACTOR * int(math.ceil(math.log(l_q))), l_q)
    counts = _sample_counts(l_q, l_k, u)
    scale = 1.0 / math.sqrt(d)
    qt = 256
    n_tiles = l_q // qt
    bh = b * h

    q_t = jnp.transpose(queries, (0, 2, 1, 3)).reshape(bh, l_q, d)
    k_t = jnp.transpose(keys, (0, 2, 1, 3)).reshape(bh, l_k, d)
    v_t = jnp.transpose(values, (0, 2, 1, 3)).reshape(bh, l_k, d)

    m = pl.pallas_call(
        functools.partial(_measure_kernel, l_k=l_k, qt=qt),
        grid=(bh,),
        in_specs=[
            pl.BlockSpec((l_q, l_k), lambda g: (0, 0)),
            pl.BlockSpec((None, l_q, d), lambda g: (g, 0, 0)),
            pl.BlockSpec((None, l_k, d), lambda g: (g, 0, 0)),
        ],
        out_specs=pl.BlockSpec((None, n_tiles, qt), lambda g: (g, 0, 0)),
        out_shape=jax.ShapeDtypeStruct((bh, n_tiles, qt), jnp.float32),
        compiler_params=pltpu.CompilerParams(
            dimension_semantics=("arbitrary",)),
    )(counts, q_t, k_t)

    sel = pl.pallas_call(
        functools.partial(_topk_kernel, n_top=u_part, l_q=l_q),
        grid=(1,),
        in_specs=[pl.BlockSpec((bh, l_q), lambda g: (0, 0))],
        out_specs=pl.BlockSpec((bh, l_q), lambda g: (0, 0)),
        out_shape=jax.ShapeDtypeStruct((bh, l_q), jnp.int32),
        scratch_shapes=[pltpu.VMEM((bh, l_q), jnp.float32)],
        compiler_params=pltpu.CompilerParams(
            dimension_semantics=("arbitrary",)),
    )(m.reshape(bh, l_q))

    out = pl.pallas_call(
        functools.partial(_attn_kernel, l_q=l_q, d=d, n_top=u_part,
                          scale=scale),
        grid=(bh,),
        in_specs=[
            pl.BlockSpec((None, 1, l_q), lambda g: (g, 0, 0)),
            pl.BlockSpec((None, l_q, d), lambda g: (g, 0, 0)),
            pl.BlockSpec((None, l_k, d), lambda g: (g, 0, 0)),
            pl.BlockSpec((None, l_k, d), lambda g: (g, 0, 0)),
        ],
        out_specs=pl.BlockSpec((None, l_q, d), lambda g: (g, 0, 0)),
        out_shape=jax.ShapeDtypeStruct((bh, l_q, d), jnp.float32),
        compiler_params=pltpu.CompilerParams(
            dimension_semantics=("arbitrary",)),
    )(sel.reshape(bh, 1, l_q), q_t, k_t, v_t)

    return jnp.transpose(out.reshape(b, h, l_q, d), (0, 2, 1, 3))


# speedup vs baseline: 2.7200x; 1.0880x over previous
import functools
import math

import jax
import jax.numpy as jnp
import numpy as np
from jax.experimental import pallas as pl
from jax.experimental.pallas import tpu as pltpu

_FACTOR = 5
_NEG_INF = float("-inf")


@functools.lru_cache(maxsize=None)
def _sample_counts_np(l_q: int, l_k: int, u: int):
    with jax.ensure_compile_time_eval():
        idx = jax.random.randint(jax.random.key(42), (l_q, u), 0, l_k)
        idx_np = np.asarray(idx)
    counts = np.zeros((l_q, l_k), np.float32)
    np.add.at(counts, (np.arange(l_q)[:, None], idx_np), 1.0)
    return counts


def _sample_counts(l_q: int, l_k: int, u: int):
    try:
        return jnp.asarray(_sample_counts_np(l_q, l_k, u))
    except Exception:
        idx = jax.random.randint(jax.random.key(42), (l_q, u), 0, l_k)
        return jnp.zeros((l_q, l_k), jnp.float32).at[
            jnp.arange(l_q)[:, None], idx].add(1.0)


def _measure_kernel(c_ref, q_ref, k_ref, m_ref, *, l_k, qt):
    q = q_ref[:, :]
    k = k_ref[:, :]
    n_tiles = q.shape[0] // qt
    for t in range(n_tiles):
        q_t = q[t * qt:(t + 1) * qt, :]
        c_t = c_ref[t * qt:(t + 1) * qt, :]
        s_t = jax.lax.dot_general(
            q_t, k, (((1,), (1,)), ((), ())),
            preferred_element_type=jnp.float32)
        s_sum = jnp.sum(s_t * c_t, axis=1)
        s_max = jnp.max(jnp.where(c_t > 0.0, s_t, _NEG_INF), axis=1)
        m_ref[t, :] = s_max - s_sum / l_k


def _topk_kernel(m_in_ref, sel_ref, m_ref, *, n_top, l_q):
    col = jax.lax.broadcasted_iota(jnp.int32, m_in_ref.shape, 1)
    m_ref[:] = m_in_ref[:]
    sel_ref[:] = jnp.full(m_in_ref.shape, -1, jnp.int32)

    def body(i, _):
        m = m_ref[:]
        mx = jnp.max(m, axis=1, keepdims=True)
        li = jnp.min(jnp.where(m == mx, col, l_q), axis=1, keepdims=True)
        hit = col == li
        sel_ref[:] = jnp.where(hit, i, sel_ref[:])
        m_ref[:] = jnp.where(hit, _NEG_INF, m)
        return 0

    jax.lax.fori_loop(0, n_top, body, 0)


def _attn_kernel(sel_ref, q_ref, k_ref, v_ref, o_ref, *, l_q, d, n_top,
                 scale):
    q = q_ref[:, :]
    k = k_ref[:, :]
    v = v_ref[:, :]

    row_iota = jax.lax.broadcasted_iota(jnp.int32, (n_top, l_q), 0)
    oh = (row_iota == sel_ref[:, :]).astype(jnp.float32)

    q_red = jax.lax.dot_general(
        oh, q, (((1,), (0,)), ((), ())),
        preferred_element_type=jnp.float32)
    scores = jax.lax.dot_general(
        q_red, k, (((1,), (1,)), ((), ())),
        preferred_element_type=jnp.float32) * scale
    s_max = jnp.max(scores, axis=1, keepdims=True)
    e = jnp.exp(scores - s_max)
    attn = e / jnp.sum(e, axis=1, keepdims=True)
    upd = jax.lax.dot_general(
        attn, v, (((1,), (0,)), ((), ())),
        preferred_element_type=jnp.float32)

    v_mean = jnp.mean(v, axis=0, keepdims=True)
    scattered = jax.lax.dot_general(
        oh, upd, (((0,), (0,)), ((), ())),
        preferred_element_type=jnp.float32)
    row_sel = jnp.sum(oh, axis=0)[:, None]
    o_ref[:, :] = jnp.where(row_sel > 0.5, scattered,
                            jnp.broadcast_to(v_mean, (l_q, d)))


def kernel(queries, keys, values):
    b, l_q, h, d = queries.shape
    l_k = keys.shape[1]
    u_part = min(_FACTOR * int(math.ceil(math.log(l_k))), l_k)
    u = min(_FACTOR * int(math.ceil(math.log(l_q))), l_q)
    counts = _sample_counts(l_q, l_k, u)
    scale = 1.0 / math.sqrt(d)
    qt = 256
    n_tiles = l_q // qt
    bh = b * h

    q_t = jnp.transpose(queries, (0, 2, 1, 3)).reshape(bh, l_q, d)
    k_t = jnp.transpose(keys, (0, 2, 1, 3)).reshape(bh, l_k, d)
    v_t = jnp.transpose(values, (0, 2, 1, 3)).reshape(bh, l_k, d)

    m = pl.pallas_call(
        functools.partial(_measure_kernel, l_k=l_k, qt=qt),
        grid=(bh,),
        in_specs=[
            pl.BlockSpec((l_q, l_k), lambda g: (0, 0)),
            pl.BlockSpec((None, l_q, d), lambda g: (g, 0, 0)),
            pl.BlockSpec((None, l_k, d), lambda g: (g, 0, 0)),
        ],
        out_specs=pl.BlockSpec((None, n_tiles, qt), lambda g: (g, 0, 0)),
        out_shape=jax.ShapeDtypeStruct((bh, n_tiles, qt), jnp.float32),
        compiler_params=pltpu.CompilerParams(
            dimension_semantics=("arbitrary",)),
    )(counts, q_t, k_t)

    sel = pl.pallas_call(
        functools.partial(_topk_kernel, n_top=u_part, l_q=l_q),
        grid=(1,),
        in_specs=[pl.BlockSpec((bh, l_q), lambda g: (0, 0))],
        out_specs=pl.BlockSpec((bh, l_q), lambda g: (0, 0)),
        out_shape=jax.ShapeDtypeStruct((bh, l_q), jnp.int32),
        scratch_shapes=[pltpu.VMEM((bh, l_q), jnp.float32)],
        compiler_params=pltpu.CompilerParams(
            dimension_semantics=("arbitrary",)),
    )(m.reshape(bh, l_q))

    out = pl.pallas_call(
        functools.partial(_attn_kernel, l_q=l_q, d=d, n_top=u_part,
                          scale=scale),
        grid=(bh,),
        in_specs=[
            pl.BlockSpec((None, 1, l_q), lambda g: (g, 0, 0)),
            pl.BlockSpec((None, l_q, d), lambda g: (g, 0, 0)),
            pl.BlockSpec((None, l_k, d), lambda g: (g, 0, 0)),
            pl.BlockSpec((None, l_k, d), lambda g: (g, 0, 0)),
        ],
        out_specs=pl.BlockSpec((None, l_q, d), lambda g: (g, 0, 0)),
        out_shape=jax.ShapeDtypeStruct((bh, l_q, d), jnp.float32),
        compiler_params=pltpu.CompilerParams(
            dimension_semantics=("arbitrary",)),
    )(sel.reshape(bh, 1, l_q), q_t, k_t, v_t)

    return jnp.transpose(out.reshape(b, h, l_q, d), (0, 2, 1, 3))
